# baseline (speedup 1.0000x reference)
import functools

import jax
import jax.numpy as jnp
import numpy as np
from jax import lax
from jax.experimental import pallas as pl
from jax.experimental.pallas import tpu as pltpu
from jax.experimental.pallas import tpu_sc as plsc

_NC = 2
_NS = 16
_KB = 4
_S = 2
_Z = np.int32(0)


def _make_sc_agg(n, d, e, cc):
    nw = _NC * _NS
    w_per = e // nw
    g = w_per // cc
    rows_t = n // _NS
    nfull = rows_t // cc
    rem = rows_t % cc

    scratch = [
        pltpu.VMEM_SHARED((n, d), jnp.float32),
        pltpu.VMEM((g, cc), jnp.int32),
        pltpu.VMEM((g, cc), jnp.int32),
        [pltpu.VMEM((cc, d), jnp.float32)] * _KB,
        [pltpu.SemaphoreType.DMA] * _KB,
        [pltpu.SemaphoreType.DMA] * _KB,
    ]
    mesh = plsc.VectorSubcoreMesh(core_axis_name="c", subcore_axis_name="s",
                                  num_cores=_NC, num_subcores=_NS)

    @functools.partial(
        pl.kernel, mesh=mesh, scratch_types=scratch,
        out_type=jax.ShapeDtypeStruct((_NC, n, d), jnp.float32),
        compiler_params=pltpu.CompilerParams(use_tc_tiling_on_sc=False))
    def body(src_hbm, dst_hbm, feat_hbm, agg_out,
             acc, sidx, didx, rowbufs, gsems, ssems):
        i32 = jnp.int32
        c = lax.axis_index("c")
        s = lax.axis_index("s")
        wid = c * i32(_NS) + s
        row0 = s * i32(rows_t)

        def _zr(r, carry):
            for k in range(d // 16):
                rowbufs[0][r, pl.ds(k * 16, 16)] = jnp.zeros((16,),
                                                             jnp.float32)
            return carry
        lax.fori_loop(jnp.int32(0), jnp.int32(cc), _zr, jnp.int32(0))
        for t in range(nfull):
            pltpu.sync_copy(rowbufs[0],
                            acc.at[pl.ds(row0 + i32(t * cc), cc)])
        if rem:
            pltpu.sync_copy(rowbufs[0].at[pl.ds(0, rem)],
                            acc.at[pl.ds(row0 + i32(nfull * cc), rem)])
        plsc.subcore_barrier()

        pltpu.sync_copy(src_hbm.at[pl.ds(wid * i32(g), g)], sidx)
        pltpu.sync_copy(dst_hbm.at[pl.ds(wid * i32(g), g)], didx)

        def _gather(j, b):
            pltpu.async_copy(feat_hbm.at[sidx.at[j]], rowbufs[b], gsems[b])

        def _wait_gather(b):
            pltpu.make_async_copy(feat_hbm.at[sidx.at[i32(0)]], rowbufs[b],
                                  gsems[b]).wait()

        def _scatter(q, b):
            pltpu.async_copy(rowbufs[b], acc.at[didx.at[q]], ssems[b],
                             add=True)

        def _wait_scatter(b):
            pltpu.make_async_copy(rowbufs[b], acc.at[didx.at[i32(0)]],
                                  ssems[b]).wait()

        def _pos(gg, carry):
            for b in range(_KB):
                j = gg * i32(_KB) + i32(b)
                pl.when((gg > 0) & (j < i32(g + _KB)))(
                    lambda b=b: _wait_scatter(b))
                pl.when(j < i32(g))(lambda j=j, b=b: _gather(j, b))
                bq = (b - _S) % _KB
                q = j - i32(_S)

                def _do(q=q, bq=bq):
                    _wait_gather(bq)
                    _scatter(q, bq)
                pl.when((q >= 0) & (q < i32(g)))(_do)
            return carry
        lax.fori_loop(jnp.int32(0), jnp.int32(g // _KB + 2), _pos,
                      jnp.int32(0))

        plsc.subcore_barrier()

        pltpu.sync_copy(acc.at[pl.ds(row0, rows_t)],
                        agg_out.at[c, pl.ds(row0, rows_t)])

    return body


def _make_sc_count(n, e, cc):
    nw = _NC * _NS
    w_per = e // nw
    g = w_per // cc
    rows_t = n // _NS

    scratch = [
        pltpu.VMEM_SHARED((n, 16), jnp.float32),
        pltpu.VMEM((rows_t, 16), jnp.float32),
        pltpu.VMEM((cc, 16), jnp.float32),
        pltpu.VMEM((g, cc), jnp.int32),
        [pltpu.SemaphoreType.DMA] * _KB,
    ]
    mesh = plsc.VectorSubcoreMesh(core_axis_name="c", subcore_axis_name="s",
                                  num_cores=_NC, num_subcores=_NS)

    @functools.partial(
        pl.kernel, mesh=mesh, scratch_types=scratch,
        out_type=jax.ShapeDtypeStruct((_NC, n, 16), jnp.float32),
        compiler_params=pltpu.CompilerParams(use_tc_tiling_on_sc=False))
    def body(dst_hbm, cnt_out, cacc, cstage, ones, didx, ssems):
        i32 = jnp.int32
        c = lax.axis_index("c")
        s = lax.axis_index("s")
        wid = c * i32(_NS) + s
        row0 = s * i32(rows_t)

        def _zc(r, carry):
            cstage[r, pl.ds(0, 16)] = jnp.zeros((16,), jnp.float32)
            return carry
        lax.fori_loop(jnp.int32(0), jnp.int32(rows_t), _zc, jnp.int32(0))
        pltpu.sync_copy(cstage, cacc.at[pl.ds(row0, rows_t)])

        def _o(r, carry):
            ones[r, pl.ds(0, 16)] = jnp.full((16,), 1.0, jnp.float32)
            return carry
        lax.fori_loop(jnp.int32(0), jnp.int32(cc), _o, jnp.int32(0))
        plsc.subcore_barrier()

        pltpu.sync_copy(dst_hbm.at[pl.ds(wid * i32(g), g)], didx)

        def _wait(b):
            pltpu.make_async_copy(ones, cacc.at[didx.at[i32(0)]],
                                  ssems[b]).wait()

        def _pos(gg, carry):
            for b in range(_KB):
                j = gg * i32(_KB) + i32(b)
                pl.when((gg > 0) & (j < i32(g + _KB)))(
                    lambda b=b: _wait(b))
                def _fire(j=j, b=b):
                    pltpu.async_copy(ones, cacc.at[didx.at[j]], ssems[b],
                                     add=True)
                pl.when(j < i32(g))(_fire)
            return carry
        lax.fori_loop(jnp.int32(0), jnp.int32(g // _KB + 2), _pos,
                      jnp.int32(0))

        plsc.subcore_barrier()
        pltpu.sync_copy(cacc.at[pl.ds(row0, rows_t)], cstage)
        pltpu.sync_copy(cstage, cnt_out.at[c, pl.ds(row0, rows_t)])

    return body


def _tc_first(p, cnt2, x, wl_t, wr_t, b8):
    n, d = x.shape
    do = wl_t.shape[1]
    blk = 1000
    grid = (n // blk,)
    one = np.int32(1)
    in_specs = [
        pl.BlockSpec((1, blk, d), lambda i: (_Z, i, _Z)),
        pl.BlockSpec((1, blk, d), lambda i: (one, i, _Z)),
        pl.BlockSpec((1, blk, 16), lambda i: (_Z, i, _Z)),
        pl.BlockSpec((1, blk, 16), lambda i: (one, i, _Z)),
        pl.BlockSpec((blk, d), lambda i: (i, _Z)),
        pl.BlockSpec(wl_t.shape, lambda i: (_Z, _Z)),
        pl.BlockSpec(wr_t.shape, lambda i: (_Z, _Z)),
        pl.BlockSpec(b8.shape, lambda i: (_Z, _Z)),
    ]

    def body(p0r, p1r, c0r, c1r, xr, wlr, wrr, br, outr, invr):
        cnt = c0r[0, :, 0:1] + c1r[0, :, 0:1]
        inv = 1.0 / jnp.maximum(cnt, 1.0)
        agg = (p0r[0] + p1r[0]) * inv
        z = (jnp.dot(agg, wlr[...], preferred_element_type=jnp.float32)
             + jnp.dot(xr[...], wrr[...], preferred_element_type=jnp.float32)
             + br[0:1, :])
        outr[...] = jnp.maximum(z, 0.0)
        invr[...] = inv

    return pl.pallas_call(
        body, grid=grid, in_specs=in_specs,
        out_specs=[pl.BlockSpec((blk, do), lambda i: (i, _Z)),
                   pl.BlockSpec((blk, 1), lambda i: (i, _Z))],
        out_shape=[jax.ShapeDtypeStruct((n, do), jnp.float32),
                   jax.ShapeDtypeStruct((n, 1), jnp.float32)])(
            p, p, cnt2, cnt2, x, wl_t, wr_t, b8)


def _tc_mid(p, inv, h, wl_t, wr_t, b8, extra_wt):
    n, d = h.shape
    do = wl_t.shape[1]
    blk = 1000
    grid = (n // blk,)
    one = np.int32(1)
    in_specs = [
        pl.BlockSpec((1, blk, d), lambda i: (_Z, i, _Z)),
        pl.BlockSpec((1, blk, d), lambda i: (one, i, _Z)),
        pl.BlockSpec((blk, 1), lambda i: (i, _Z)),
        pl.BlockSpec((blk, d), lambda i: (i, _Z)),
        pl.BlockSpec(wl_t.shape, lambda i: (_Z, _Z)),
        pl.BlockSpec(wr_t.shape, lambda i: (_Z, _Z)),
        pl.BlockSpec(b8.shape, lambda i: (_Z, _Z)),
        pl.BlockSpec(extra_wt.shape, lambda i: (_Z, _Z)),
    ]

    def body(p0r, p1r, invr, hr, wlr, wrr, br, ewr, outr, yr):
        agg = (p0r[0] + p1r[0]) * invr[...]
        z = (jnp.dot(agg, wlr[...], preferred_element_type=jnp.float32)
             + jnp.dot(hr[...], wrr[...], preferred_element_type=jnp.float32)
             + br[0:1, :])
        z = jnp.maximum(z, 0.0)
        outr[...] = z
        yr[...] = jnp.dot(z, ewr[...], preferred_element_type=jnp.float32)

    return pl.pallas_call(
        body, grid=grid, in_specs=in_specs,
        out_specs=[pl.BlockSpec((blk, do), lambda i: (i, _Z)),
                   pl.BlockSpec((blk, extra_wt.shape[1]),
                                lambda i: (i, _Z))],
        out_shape=[jax.ShapeDtypeStruct((n, do), jnp.float32),
                   jax.ShapeDtypeStruct((n, extra_wt.shape[1]),
                                        jnp.float32)])(
            p, p, inv, h, wl_t, wr_t, b8, extra_wt)


def _tc_final(p, inv, h, wr_t, b8):
    n, d = h.shape
    do = wr_t.shape[1]
    blk = 1000
    grid = (n // blk,)
    one = np.int32(1)
    in_specs = [
        pl.BlockSpec((1, blk, do), lambda i: (_Z, i, _Z)),
        pl.BlockSpec((1, blk, do), lambda i: (one, i, _Z)),
        pl.BlockSpec((blk, 1), lambda i: (i, _Z)),
        pl.BlockSpec((blk, d), lambda i: (i, _Z)),
        pl.BlockSpec(wr_t.shape, lambda i: (_Z, _Z)),
        pl.BlockSpec(b8.shape, lambda i: (_Z, _Z)),
    ]

    def body(p0r, p1r, invr, hr, wrr, br, outr):
        agg = (p0r[0] + p1r[0]) * invr[...]
        outr[...] = (agg
                     + jnp.dot(hr[...], wrr[...],
                               preferred_element_type=jnp.float32)
                     + br[0:1, :])

    return pl.pallas_call(
        body, grid=grid, in_specs=in_specs,
        out_specs=pl.BlockSpec((blk, do), lambda i: (i, _Z)),
        out_shape=jax.ShapeDtypeStruct((n, do), jnp.float32))(
            p, p, inv, h, wr_t, b8)


def kernel(x, edge_index, W1l, b1l, W1r, W2l, b2l, W2r, W3l, b3l, W3r,
           g1, be1, g2, be2, g3, be3):
    n, d_in = x.shape
    e = edge_index.shape[1]
    d_h = W1l.shape[0]
    d_out = W3l.shape[0]
    src = edge_index[0].astype(jnp.int32)
    dst = edge_index[1].astype(jnp.int32)

    src40 = src.reshape(e // 40, 40)
    dst40 = dst.reshape(e // 40, 40)
    src80 = src.reshape(e // 80, 80)
    dst80 = dst.reshape(e // 80, 80)
    x = x.astype(jnp.float32)

    inv_std = np.float32(1.0 / np.sqrt(1.0 + 1e-5))

    def fold(wl, bl, wr, gamma, beta):
        sc = gamma * inv_std
        wl_t = wl.T * sc[None, :]
        wr_t = wr.T * sc[None, :]
        b8 = jnp.broadcast_to((bl * sc + beta)[None, :], (8, sc.shape[0]))
        return wl_t, wr_t, b8

    w1l_t, w1r_t, b1_8 = fold(W1l, b1l, W1r, g1, be1)
    w2l_t, w2r_t, b2_8 = fold(W2l, b2l, W2r, g2, be2)
    w3l_t, w3r_t, b3_8 = fold(W3l, b3l, W3r, g3, be3)

    cnt2 = _make_sc_count(n, e, 40)(dst40)
    agg1 = _make_sc_agg(n, d_in, e, 40)(src40, dst40, x)
    h1, inv = _tc_first(agg1, cnt2, x, w1l_t, w1r_t, b1_8)
    agg2 = _make_sc_agg(n, d_h, e, 40)(src40, dst40, h1)
    h2, y3 = _tc_mid(agg2, inv, h1, w2l_t, w2r_t, b2_8, w3l_t)
    agg3 = _make_sc_agg(n, d_out, e, 80)(src80, dst80, y3)
    return _tc_final(agg3, inv, h2, w3r_t, b3_8)


# speedup vs baseline: 1.0399x; 1.0399x over previous
import functools

import jax
import jax.numpy as jnp
import numpy as np
from jax import lax
from jax.experimental import pallas as pl
from jax.experimental.pallas import tpu as pltpu
from jax.experimental.pallas import tpu_sc as plsc

_NC = 2
_NS = 16
_KB = 4
_S = 2
_Z = np.int32(0)


def _make_sc_agg(n, d, e, cc, kb=_KB, lag=_S):
    nw = _NC * _NS
    w_per = e // nw
    g = w_per // cc
    rows_t = n // _NS
    nfull = rows_t // cc
    rem = rows_t % cc

    scratch = [
        pltpu.VMEM_SHARED((n, d), jnp.float32),
        pltpu.VMEM((g, cc), jnp.int32),
        pltpu.VMEM((g, cc), jnp.int32),
        [pltpu.VMEM((cc, d), jnp.float32)] * kb,
        [pltpu.SemaphoreType.DMA] * kb,
        [pltpu.SemaphoreType.DMA] * kb,
    ]
    mesh = plsc.VectorSubcoreMesh(core_axis_name="c", subcore_axis_name="s",
                                  num_cores=_NC, num_subcores=_NS)

    @functools.partial(
        pl.kernel, mesh=mesh, scratch_types=scratch,
        out_type=jax.ShapeDtypeStruct((_NC, n, d), jnp.float32),
        compiler_params=pltpu.CompilerParams(use_tc_tiling_on_sc=False))
    def body(src_hbm, dst_hbm, feat_hbm, agg_out,
             acc, sidx, didx, rowbufs, gsems, ssems):
        i32 = jnp.int32
        c = lax.axis_index("c")
        s = lax.axis_index("s")
        wid = c * i32(_NS) + s
        row0 = s * i32(rows_t)

        def _zr(r, carry):
            for k in range(d // 16):
                rowbufs[0][r, pl.ds(k * 16, 16)] = jnp.zeros((16,),
                                                             jnp.float32)
            return carry
        lax.fori_loop(jnp.int32(0), jnp.int32(cc), _zr, jnp.int32(0))
        for t in range(nfull):
            pltpu.sync_copy(rowbufs[0],
                            acc.at[pl.ds(row0 + i32(t * cc), cc)])
        if rem:
            pltpu.sync_copy(rowbufs[0].at[pl.ds(0, rem)],
                            acc.at[pl.ds(row0 + i32(nfull * cc), rem)])
        plsc.subcore_barrier()

        pltpu.sync_copy(src_hbm.at[pl.ds(wid * i32(g), g)], sidx)
        pltpu.sync_copy(dst_hbm.at[pl.ds(wid * i32(g), g)], didx)

        def _gather(j, b):
            pltpu.async_copy(feat_hbm.at[sidx.at[j]], rowbufs[b], gsems[b])

        def _wait_gather(b):
            pltpu.make_async_copy(feat_hbm.at[sidx.at[i32(0)]], rowbufs[b],
                                  gsems[b]).wait()

        def _scatter(q, b):
            pltpu.async_copy(rowbufs[b], acc.at[didx.at[q]], ssems[b],
                             add=True)

        def _wait_scatter(b):
            pltpu.make_async_copy(rowbufs[b], acc.at[didx.at[i32(0)]],
                                  ssems[b]).wait()

        def _pos(gg, carry):
            for b in range(kb):
                j = gg * i32(kb) + i32(b)
                pl.when((gg > 0) & (j < i32(g + kb)))(
                    lambda b=b: _wait_scatter(b))
                pl.when(j < i32(g))(lambda j=j, b=b: _gather(j, b))
                bq = (b - lag) % kb
                q = j - i32(lag)

                def _do(q=q, bq=bq):
                    _wait_gather(bq)
                    _scatter(q, bq)
                pl.when((q >= 0) & (q < i32(g)))(_do)
            return carry
        lax.fori_loop(jnp.int32(0), jnp.int32(g // kb + 2), _pos,
                      jnp.int32(0))

        plsc.subcore_barrier()

        pltpu.sync_copy(acc.at[pl.ds(row0, rows_t)],
                        agg_out.at[c, pl.ds(row0, rows_t)])

    return body


def _make_sc_count(n, e, cc):
    nw = _NC * _NS
    w_per = e // nw
    g = w_per // cc
    rows_t = n // _NS

    scratch = [
        pltpu.VMEM_SHARED((n, 16), jnp.float32),
        pltpu.VMEM((rows_t, 16), jnp.float32),
        pltpu.VMEM((cc, 16), jnp.float32),
        pltpu.VMEM((g, cc), jnp.int32),
        [pltpu.SemaphoreType.DMA] * _KB,
    ]
    mesh = plsc.VectorSubcoreMesh(core_axis_name="c", subcore_axis_name="s",
                                  num_cores=_NC, num_subcores=_NS)

    @functools.partial(
        pl.kernel, mesh=mesh, scratch_types=scratch,
        out_type=jax.ShapeDtypeStruct((_NC, n, 16), jnp.float32),
        compiler_params=pltpu.CompilerParams(use_tc_tiling_on_sc=False))
    def body(dst_hbm, cnt_out, cacc, cstage, ones, didx, ssems):
        i32 = jnp.int32
        c = lax.axis_index("c")
        s = lax.axis_index("s")
        wid = c * i32(_NS) + s
        row0 = s * i32(rows_t)

        def _zc(r, carry):
            cstage[r, pl.ds(0, 16)] = jnp.zeros((16,), jnp.float32)
            return carry
        lax.fori_loop(jnp.int32(0), jnp.int32(rows_t), _zc, jnp.int32(0))
        pltpu.sync_copy(cstage, cacc.at[pl.ds(row0, rows_t)])

        def _o(r, carry):
            ones[r, pl.ds(0, 16)] = jnp.full((16,), 1.0, jnp.float32)
            return carry
        lax.fori_loop(jnp.int32(0), jnp.int32(cc), _o, jnp.int32(0))
        plsc.subcore_barrier()

        pltpu.sync_copy(dst_hbm.at[pl.ds(wid * i32(g), g)], didx)

        def _wait(b):
            pltpu.make_async_copy(ones, cacc.at[didx.at[i32(0)]],
                                  ssems[b]).wait()

        def _pos(gg, carry):
            for b in range(_KB):
                j = gg * i32(_KB) + i32(b)
                pl.when((gg > 0) & (j < i32(g + _KB)))(
                    lambda b=b: _wait(b))
                def _fire(j=j, b=b):
                    pltpu.async_copy(ones, cacc.at[didx.at[j]], ssems[b],
                                     add=True)
                pl.when(j < i32(g))(_fire)
            return carry
        lax.fori_loop(jnp.int32(0), jnp.int32(g // _KB + 2), _pos,
                      jnp.int32(0))

        plsc.subcore_barrier()
        pltpu.sync_copy(cacc.at[pl.ds(row0, rows_t)], cstage)
        pltpu.sync_copy(cstage, cnt_out.at[c, pl.ds(row0, rows_t)])

    return body


def _tc_first(p, cnt2, x, wl_t, wr_t, b8):
    n, d = x.shape
    do = wl_t.shape[1]
    blk = 1000
    grid = (n // blk,)
    one = np.int32(1)
    in_specs = [
        pl.BlockSpec((1, blk, d), lambda i: (_Z, i, _Z)),
        pl.BlockSpec((1, blk, d), lambda i: (one, i, _Z)),
        pl.BlockSpec((1, blk, 16), lambda i: (_Z, i, _Z)),
        pl.BlockSpec((1, blk, 16), lambda i: (one, i, _Z)),
        pl.BlockSpec((blk, d), lambda i: (i, _Z)),
        pl.BlockSpec(wl_t.shape, lambda i: (_Z, _Z)),
        pl.BlockSpec(wr_t.shape, lambda i: (_Z, _Z)),
        pl.BlockSpec(b8.shape, lambda i: (_Z, _Z)),
    ]

    def body(p0r, p1r, c0r, c1r, xr, wlr, wrr, br, outr, invr):
        cnt = c0r[0, :, 0:1] + c1r[0, :, 0:1]
        inv = 1.0 / jnp.maximum(cnt, 1.0)
        agg = (p0r[0] + p1r[0]) * inv
        z = (jnp.dot(agg, wlr[...], preferred_element_type=jnp.float32)
             + jnp.dot(xr[...], wrr[...], preferred_element_type=jnp.float32)
             + br[0:1, :])
        outr[...] = jnp.maximum(z, 0.0)
        invr[...] = inv

    return pl.pallas_call(
        body, grid=grid, in_specs=in_specs,
        out_specs=[pl.BlockSpec((blk, do), lambda i: (i, _Z)),
                   pl.BlockSpec((blk, 1), lambda i: (i, _Z))],
        out_shape=[jax.ShapeDtypeStruct((n, do), jnp.float32),
                   jax.ShapeDtypeStruct((n, 1), jnp.float32)])(
            p, p, cnt2, cnt2, x, wl_t, wr_t, b8)


def _tc_mid(p, inv, h, wl_t, wr_t, b8, extra_wt):
    n, d = h.shape
    do = wl_t.shape[1]
    blk = 1000
    grid = (n // blk,)
    one = np.int32(1)
    in_specs = [
        pl.BlockSpec((1, blk, d), lambda i: (_Z, i, _Z)),
        pl.BlockSpec((1, blk, d), lambda i: (one, i, _Z)),
        pl.BlockSpec((blk, 1), lambda i: (i, _Z)),
        pl.BlockSpec((blk, d), lambda i: (i, _Z)),
        pl.BlockSpec(wl_t.shape, lambda i: (_Z, _Z)),
        pl.BlockSpec(wr_t.shape, lambda i: (_Z, _Z)),
        pl.BlockSpec(b8.shape, lambda i: (_Z, _Z)),
        pl.BlockSpec(extra_wt.shape, lambda i: (_Z, _Z)),
    ]

    def body(p0r, p1r, invr, hr, wlr, wrr, br, ewr, outr, yr):
        agg = (p0r[0] + p1r[0]) * invr[...]
        z = (jnp.dot(agg, wlr[...], preferred_element_type=jnp.float32)
             + jnp.dot(hr[...], wrr[...], preferred_element_type=jnp.float32)
             + br[0:1, :])
        z = jnp.maximum(z, 0.0)
        outr[...] = z
        yr[...] = jnp.dot(z, ewr[...], preferred_element_type=jnp.float32)

    return pl.pallas_call(
        body, grid=grid, in_specs=in_specs,
        out_specs=[pl.BlockSpec((blk, do), lambda i: (i, _Z)),
                   pl.BlockSpec((blk, extra_wt.shape[1]),
                                lambda i: (i, _Z))],
        out_shape=[jax.ShapeDtypeStruct((n, do), jnp.float32),
                   jax.ShapeDtypeStruct((n, extra_wt.shape[1]),
                                        jnp.float32)])(
            p, p, inv, h, wl_t, wr_t, b8, extra_wt)


def _tc_final(p, inv, h, wr_t, b8):
    n, d = h.shape
    do = wr_t.shape[1]
    blk = 1000
    grid = (n // blk,)
    one = np.int32(1)
    in_specs = [
        pl.BlockSpec((1, blk, do), lambda i: (_Z, i, _Z)),
        pl.BlockSpec((1, blk, do), lambda i: (one, i, _Z)),
        pl.BlockSpec((blk, 1), lambda i: (i, _Z)),
        pl.BlockSpec((blk, d), lambda i: (i, _Z)),
        pl.BlockSpec(wr_t.shape, lambda i: (_Z, _Z)),
        pl.BlockSpec(b8.shape, lambda i: (_Z, _Z)),
    ]

    def body(p0r, p1r, invr, hr, wrr, br, outr):
        agg = (p0r[0] + p1r[0]) * invr[...]
        outr[...] = (agg
                     + jnp.dot(hr[...], wrr[...],
                               preferred_element_type=jnp.float32)
                     + br[0:1, :])

    return pl.pallas_call(
        body, grid=grid, in_specs=in_specs,
        out_specs=pl.BlockSpec((blk, do), lambda i: (i, _Z)),
        out_shape=jax.ShapeDtypeStruct((n, do), jnp.float32))(
            p, p, inv, h, wr_t, b8)


def kernel(x, edge_index, W1l, b1l, W1r, W2l, b2l, W2r, W3l, b3l, W3r,
           g1, be1, g2, be2, g3, be3):
    n, d_in = x.shape
    e = edge_index.shape[1]
    d_h = W1l.shape[0]
    d_out = W3l.shape[0]
    src = edge_index[0].astype(jnp.int32)
    dst = edge_index[1].astype(jnp.int32)

    src40 = src.reshape(e // 40, 40)
    dst40 = dst.reshape(e // 40, 40)
    src80 = src.reshape(e // 80, 80)
    dst80 = dst.reshape(e // 80, 80)
    x = x.astype(jnp.float32)

    inv_std = np.float32(1.0 / np.sqrt(1.0 + 1e-5))

    def fold(wl, bl, wr, gamma, beta):
        sc = gamma * inv_std
        wl_t = wl.T * sc[None, :]
        wr_t = wr.T * sc[None, :]
        b8 = jnp.broadcast_to((bl * sc + beta)[None, :], (8, sc.shape[0]))
        return wl_t, wr_t, b8

    w1l_t, w1r_t, b1_8 = fold(W1l, b1l, W1r, g1, be1)
    w2l_t, w2r_t, b2_8 = fold(W2l, b2l, W2r, g2, be2)
    w3l_t, w3r_t, b3_8 = fold(W3l, b3l, W3r, g3, be3)

    cnt2 = _make_sc_count(n, e, 40)(dst40)
    agg1 = _make_sc_agg(n, d_in, e, 80, kb=3, lag=1)(src80, dst80, x)
    h1, inv = _tc_first(agg1, cnt2, x, w1l_t, w1r_t, b1_8)
    agg2 = _make_sc_agg(n, d_h, e, 80, kb=3, lag=1)(src80, dst80, h1)
    h2, y3 = _tc_mid(agg2, inv, h1, w2l_t, w2r_t, b2_8, w3l_t)
    agg3 = _make_sc_agg(n, d_out, e, 80)(src80, dst80, y3)
    return _tc_final(agg3, inv, h2, w3r_t, b3_8)
